# baseline (device time: 135932 ns/iter reference)
import jax
import jax.numpy as jnp
from jax import lax
from jax.experimental import pallas as pl
from jax.experimental.pallas import tpu as pltpu

NC = 8


def kernel(x, W):
    m, k = x.shape
    _, n_half = W.shape
    n_total = 2 * n_half
    ch = m // NC

    xb = x.astype(jnp.bfloat16)
    wb = W.astype(jnp.bfloat16)

    def body(
        x_ref, w_ref, out_ref, send_buf, recv_buf, stage,
        y_send_sems, y_recv_sems, out_sems,
    ):
        my_x = lax.axis_index("x")
        my_y = lax.axis_index("y")
        nbr_y = (my_x, 1 - my_y)

        barrier = pltpu.get_barrier_semaphore()
        pl.semaphore_signal(
            barrier, inc=1, device_id=nbr_y, device_id_type=pl.DeviceIdType.MESH
        )
        pl.semaphore_wait(barrier, 1)

        y_rdmas = []
        for c in range(NC):
            r = pl.ds(c * ch, ch)
            blk = jnp.dot(
                x_ref[r, :], w_ref[:, :], preferred_element_type=jnp.float32
            )
            send_buf[r, :] = blk.astype(jnp.bfloat16)
            rdma = pltpu.make_async_remote_copy(
                src_ref=send_buf.at[r],
                dst_ref=recv_buf.at[r],
                send_sem=y_send_sems.at[c],
                recv_sem=y_recv_sems.at[c],
                device_id=nbr_y,
                device_id_type=pl.DeviceIdType.MESH,
            )
            rdma.start()
            y_rdmas.append(rdma)

        own_start = my_y * n_half
        oth_start = (1 - my_y) * n_half
        copies = []

        for bi in range(NC):
            r = pl.ds(bi * ch, ch)
            slot = bi % 2
            if bi >= 2:
                copies[bi - 2].wait()
            y_rdmas[bi].wait_recv()
            own = send_buf[r, :].astype(jnp.float32)
            oth = recv_buf[r, :].astype(jnp.float32)
            mx = jnp.maximum(
                jnp.max(own, axis=-1, keepdims=True),
                jnp.max(oth, axis=-1, keepdims=True),
            )
            e_own = jnp.exp(own - mx)
            e_oth = jnp.exp(oth - mx)
            denom = jnp.sum(e_own, axis=-1, keepdims=True) + jnp.sum(
                e_oth, axis=-1, keepdims=True
            )
            stage[slot, :, pl.ds(own_start, n_half)] = e_own / denom
            stage[slot, :, pl.ds(oth_start, n_half)] = e_oth / denom
            cp = pltpu.make_async_copy(
                stage.at[slot], out_ref.at[r], out_sems.at[slot]
            )
            cp.start()
            copies.append(cp)

        for rdma in y_rdmas:
            rdma.wait_send()
        for cp in copies[-2:]:
            cp.wait()

    return pl.pallas_call(
        body,
        out_shape=jax.ShapeDtypeStruct((m, n_total), jnp.float32),
        in_specs=[
            pl.BlockSpec(memory_space=pltpu.VMEM),
            pl.BlockSpec(memory_space=pltpu.VMEM),
        ],
        out_specs=pl.BlockSpec(memory_space=pl.ANY),
        scratch_shapes=[
            pltpu.VMEM((m, n_half), jnp.bfloat16),
            pltpu.VMEM((m, n_half), jnp.bfloat16),
            pltpu.VMEM((2, ch, n_total), jnp.float32),
            pltpu.SemaphoreType.DMA((NC,)),
            pltpu.SemaphoreType.DMA((NC,)),
            pltpu.SemaphoreType.DMA((2,)),
        ],
        compiler_params=pltpu.CompilerParams(collective_id=0),
    )(xb, wb)


# device time: 128576 ns/iter; 1.0572x vs baseline; 1.0572x over previous
import jax
import jax.numpy as jnp
from jax import lax
from jax.experimental import pallas as pl
from jax.experimental.pallas import tpu as pltpu

NC = 8
GEMM_BLK = 128


def kernel(x, W):
    m, k = x.shape
    _, n_half = W.shape
    n_total = 2 * n_half
    half_m = m // 2
    ch = half_m // NC

    xb = x.astype(jnp.bfloat16)
    wb = W.astype(jnp.bfloat16)
    out_buf = jnp.zeros((m, n_total), jnp.float32)

    def body(
        x_ref, w_ref, dummy_ref, out_ref, send_buf, recv_buf, stage,
        y_send_sems, y_recv_sems, x_send_sems, x_recv_sems, out_sems,
    ):
        my_x = lax.axis_index("x")
        my_y = lax.axis_index("y")
        nbr_y = (my_x, 1 - my_y)
        nbr_x = (1 - my_x, my_y)
        p0 = my_x * half_m
        q0 = (1 - my_x) * half_m

        barrier = pltpu.get_barrier_semaphore()
        for nbr in (nbr_y, nbr_x):
            pl.semaphore_signal(
                barrier, inc=1, device_id=nbr, device_id_type=pl.DeviceIdType.MESH
            )
        pl.semaphore_wait(barrier, 2)

        y_rdmas = []
        for c in range(NC):
            r = pl.ds(p0 + c * ch, ch)
            blk = jnp.dot(
                x_ref[r, :], w_ref[:, :], preferred_element_type=jnp.float32
            )
            send_buf[r, :] = blk.astype(jnp.bfloat16)
            rdma = pltpu.make_async_remote_copy(
                src_ref=send_buf.at[r],
                dst_ref=recv_buf.at[r],
                send_sem=y_send_sems.at[c],
                recv_sem=y_recv_sems.at[c],
                device_id=nbr_y,
                device_id_type=pl.DeviceIdType.MESH,
            )
            rdma.start()
            y_rdmas.append(rdma)

        for i in range(0, half_m, GEMM_BLK):
            r = pl.ds(q0 + i, GEMM_BLK)
            blk = jnp.dot(
                x_ref[r, :], w_ref[:, :], preferred_element_type=jnp.float32
            )
            send_buf[r, :] = blk.astype(jnp.bfloat16)

        own_start = my_y * n_half
        oth_start = (1 - my_y) * n_half
        copies = []
        x_rdmas = []
        bi = [0]

        def softmax_block(r):
            slot = bi[0] % 2
            if bi[0] >= 2:
                copies[bi[0] - 2].wait()
            own = send_buf[r, :].astype(jnp.float32)
            oth = recv_buf[r, :].astype(jnp.float32)
            mx = jnp.maximum(
                jnp.max(own, axis=-1, keepdims=True),
                jnp.max(oth, axis=-1, keepdims=True),
            )
            e_own = jnp.exp(own - mx)
            e_oth = jnp.exp(oth - mx)
            inv = 1.0 / (
                jnp.sum(e_own, axis=-1, keepdims=True)
                + jnp.sum(e_oth, axis=-1, keepdims=True)
            )
            stage[slot, :, pl.ds(own_start, n_half)] = e_own * inv
            stage[slot, :, pl.ds(oth_start, n_half)] = e_oth * inv
            cp = pltpu.make_async_copy(
                stage.at[slot], out_ref.at[r], out_sems.at[slot]
            )
            cp.start()
            copies.append(cp)
            bi[0] += 1

        for c in range(NC):
            r = pl.ds(p0 + c * ch, ch)
            y_rdmas[c].wait_recv()
            fwd = pltpu.make_async_remote_copy(
                src_ref=recv_buf.at[r],
                dst_ref=recv_buf.at[r],
                send_sem=x_send_sems.at[c],
                recv_sem=x_recv_sems.at[c],
                device_id=nbr_x,
                device_id_type=pl.DeviceIdType.MESH,
            )
            fwd.start()
            x_rdmas.append(fwd)
            softmax_block(r)
            if c >= 1:
                x_rdmas[c - 1].wait_recv()
                softmax_block(pl.ds(q0 + (c - 1) * ch, ch))

        x_rdmas[NC - 1].wait_recv()
        softmax_block(pl.ds(q0 + (NC - 1) * ch, ch))

        for rdma in y_rdmas:
            rdma.wait_send()
        for rdma in x_rdmas:
            rdma.wait_send()
        for cp in copies[-2:]:
            cp.wait()

    return pl.pallas_call(
        body,
        out_shape=jax.ShapeDtypeStruct((m, n_total), jnp.float32),
        in_specs=[
            pl.BlockSpec(memory_space=pltpu.VMEM),
            pl.BlockSpec(memory_space=pltpu.VMEM),
            pl.BlockSpec(memory_space=pl.ANY),
        ],
        out_specs=pl.BlockSpec(memory_space=pl.ANY),
        input_output_aliases={2: 0},
        scratch_shapes=[
            pltpu.VMEM((m, n_half), jnp.bfloat16),
            pltpu.VMEM((m, n_half), jnp.bfloat16),
            pltpu.VMEM((2, ch, n_total), jnp.float32),
            pltpu.SemaphoreType.DMA((NC,)),
            pltpu.SemaphoreType.DMA((NC,)),
            pltpu.SemaphoreType.DMA((NC,)),
            pltpu.SemaphoreType.DMA((NC,)),
            pltpu.SemaphoreType.DMA((2,)),
        ],
        compiler_params=pltpu.CompilerParams(collective_id=0),
    )(xb, wb, out_buf)


# device time: 104686 ns/iter; 1.2985x vs baseline; 1.2282x over previous
import jax
import jax.numpy as jnp
from jax import lax
from jax.experimental import pallas as pl
from jax.experimental.pallas import tpu as pltpu

NC = 4
GEMM_BLK = 128


def kernel(x, W):
    m, k = x.shape
    _, n_half = W.shape
    n_total = 2 * n_half
    half_m = m // 2
    ch = half_m // NC

    xb = x.astype(jnp.bfloat16)
    wb = W.astype(jnp.bfloat16)

    def body(
        x_ref, w_ref, out_ref, send_buf, recv_buf, stage,
        y_send_sems, y_recv_sems, x_send_sems, x_recv_sems, out_sems,
    ):
        my_x = lax.axis_index("x")
        my_y = lax.axis_index("y")
        nbr_y = (my_x, 1 - my_y)
        nbr_x = (1 - my_x, my_y)
        p0 = my_x * half_m
        q0 = (1 - my_x) * half_m

        barrier = pltpu.get_barrier_semaphore()
        for nbr in (nbr_y, nbr_x):
            pl.semaphore_signal(
                barrier, inc=1, device_id=nbr, device_id_type=pl.DeviceIdType.MESH
            )
        pl.semaphore_wait(barrier, 2)

        y_rdmas = []
        x_rdmas = []
        for c in range(NC):
            r = pl.ds(p0 + c * ch, ch)
            blk = jnp.dot(
                x_ref[r, :], w_ref[:, :], preferred_element_type=jnp.float32
            )
            send_buf[r, :] = blk.astype(jnp.bfloat16)
            rdma = pltpu.make_async_remote_copy(
                src_ref=send_buf.at[r],
                dst_ref=recv_buf.at[r],
                send_sem=y_send_sems.at[c],
                recv_sem=y_recv_sems.at[c],
                device_id=nbr_y,
                device_id_type=pl.DeviceIdType.MESH,
            )
            rdma.start()
            y_rdmas.append(rdma)

        for c in range(NC):
            r = pl.ds(q0 + c * ch, ch)
            blk = jnp.dot(
                x_ref[r, :], w_ref[:, :], preferred_element_type=jnp.float32
            )
            send_buf[r, :] = blk.astype(jnp.bfloat16)
            fwd = pltpu.make_async_remote_copy(
                src_ref=send_buf.at[r],
                dst_ref=recv_buf.at[r],
                send_sem=x_send_sems.at[c],
                recv_sem=x_recv_sems.at[c],
                device_id=nbr_x,
                device_id_type=pl.DeviceIdType.MESH,
            )
            fwd.start()
            x_rdmas.append(fwd)

        own_start = my_y * n_half
        oth_start = (1 - my_y) * n_half
        copies = []
        bi = [0]

        def softmax_block(r):
            slot = bi[0] % 2
            if bi[0] >= 2:
                copies[bi[0] - 2].wait()
            own = send_buf[r, :].astype(jnp.float32)
            oth = recv_buf[r, :].astype(jnp.float32)
            mx = jnp.maximum(
                jnp.max(own, axis=-1, keepdims=True),
                jnp.max(oth, axis=-1, keepdims=True),
            )
            e_own = jnp.exp(own - mx)
            e_oth = jnp.exp(oth - mx)
            inv = 1.0 / (
                jnp.sum(e_own, axis=-1, keepdims=True)
                + jnp.sum(e_oth, axis=-1, keepdims=True)
            )
            stage[slot, :, pl.ds(own_start, n_half)] = e_own * inv
            stage[slot, :, pl.ds(oth_start, n_half)] = e_oth * inv
            cp = pltpu.make_async_copy(
                stage.at[slot], out_ref.at[r], out_sems.at[slot]
            )
            cp.start()
            copies.append(cp)
            bi[0] += 1

        for c in range(NC):
            y_rdmas[c].wait_recv()
            softmax_block(pl.ds(p0 + c * ch, ch))
            x_rdmas[c].wait_recv()
            softmax_block(pl.ds(q0 + c * ch, ch))

        for rdma in y_rdmas:
            rdma.wait_send()
        for rdma in x_rdmas:
            rdma.wait_send()
        for cp in copies[-2:]:
            cp.wait()

    return pl.pallas_call(
        body,
        out_shape=jax.ShapeDtypeStruct((m, n_total), jnp.float32),
        in_specs=[
            pl.BlockSpec(memory_space=pltpu.VMEM),
            pl.BlockSpec(memory_space=pltpu.VMEM),
        ],
        out_specs=pl.BlockSpec(memory_space=pl.ANY),
        scratch_shapes=[
            pltpu.VMEM((m, n_half), jnp.bfloat16),
            pltpu.VMEM((m, n_half), jnp.bfloat16),
            pltpu.VMEM((2, ch, n_total), jnp.float32),
            pltpu.SemaphoreType.DMA((NC,)),
            pltpu.SemaphoreType.DMA((NC,)),
            pltpu.SemaphoreType.DMA((NC,)),
            pltpu.SemaphoreType.DMA((NC,)),
            pltpu.SemaphoreType.DMA((2,)),
        ],
        compiler_params=pltpu.CompilerParams(collective_id=0),
    )(xb, wb)
